# baseline (speedup 1.0000x reference)
import functools

import jax
import jax.numpy as jnp
from jax import lax
from jax.experimental import pallas as pl
from jax.experimental.pallas import tpu as pltpu
from jax.experimental.pallas import tpu_sc as plsc

_L = 16


def _proj_body(z_ref, w1a_ref, w1b_ref, b1_ref, a_ref, b_ref):
    z = z_ref[...]
    a_ref[...] = jnp.dot(z, w1a_ref[...], precision=lax.Precision.HIGHEST,
                         preferred_element_type=jnp.float32)
    b_ref[...] = (jnp.dot(z, w1b_ref[...], precision=lax.Precision.HIGHEST,
                          preferred_element_type=jnp.float32)
                  + b1_ref[...])


def _node_projections(z, W1, b1):
    n, h = z.shape
    blk = 1000
    grid = n // blk
    return pl.pallas_call(
        _proj_body,
        grid=(grid,),
        in_specs=[
            pl.BlockSpec((blk, h), lambda i: (i, 0)),
            pl.BlockSpec((h, h), lambda i: (0, 0)),
            pl.BlockSpec((h, h), lambda i: (0, 0)),
            pl.BlockSpec((1, h), lambda i: (0, 0)),
        ],
        out_specs=[
            pl.BlockSpec((blk, h), lambda i: (i, 0)),
            pl.BlockSpec((blk, h), lambda i: (i, 0)),
        ],
        out_shape=[
            jax.ShapeDtypeStruct((n, h), jnp.float32),
            jax.ShapeDtypeStruct((n, h), jnp.float32),
        ],
    )(z, W1[:h], W1[h:], b1.reshape(1, h))


_NBUF = 4
_CHUNK = 80


def _edge_kernel(a_hbm, b_hbm, row_hbm, col_hbm, w2_hbm, b2_hbm, out_hbm,
                 idx_row, idx_col, arows, brows, w2_v, b2_v, out_v,
                 *sems, nc, epw, h):
    cid = lax.axis_index("c")
    sid = lax.axis_index("s")
    wid = sid * nc + cid
    base = wid * epw
    nchunk = epw // _CHUNK
    sem_a = sems[:_NBUF]
    sem_b = sems[_NBUF:]

    pltpu.sync_copy(w2_hbm, w2_v)
    pltpu.sync_copy(b2_hbm, b2_v)
    pltpu.sync_copy(row_hbm.at[pl.ds(base, epw)], idx_row)
    pltpu.sync_copy(col_hbm.at[pl.ds(base, epw)], idx_col)


    lanes = lax.iota(jnp.int32, _L)
    lane0 = lanes == 0
    perms = [lanes ^ (1 << p) for p in range(4)]
    acc0 = jnp.where(lane0, b2_v[...], jnp.zeros((_L,), jnp.float32))
    w2s = [w2_v[pl.ds(s * _L, _L)] for s in range(h // _L)]

    def issue(g, slot):
        off = g * _CHUNK
        for j in range(_CHUNK // _L):
            iva = idx_row[pl.ds(off + j * _L, _L)]
            ivb = idx_col[pl.ds(off + j * _L, _L)]
            pltpu.async_copy(a_hbm.at[iva],
                             arows.at[slot].at[pl.ds(j * _L, _L)],
                             sem_a[slot])
            pltpu.async_copy(b_hbm.at[ivb],
                             brows.at[slot].at[pl.ds(j * _L, _L)],
                             sem_b[slot])

    def wait(slot):
        pltpu.make_async_copy(a_hbm.at[pl.ds(0, _CHUNK)], arows.at[slot],
                              sem_a[slot]).wait()
        pltpu.make_async_copy(b_hbm.at[pl.ds(0, _CHUNK)], brows.at[slot],
                              sem_b[slot]).wait()

    def compute(g, slot):
        ar = arows.at[slot]
        br = brows.at[slot]
        gout = g * _CHUNK

        @plsc.parallel_loop(0, _CHUNK, unroll=4)
        def edge_body(e):
            ps = []
            for s in range(h // _L):
                av = ar[e, pl.ds(s * _L, _L)]
                bv = br[e, pl.ds(s * _L, _L)]
                m = jnp.maximum(av + bv, 0.0)
                ps.append(m * w2s[s])
            while len(ps) > 1:
                ps = [ps[i] + ps[i + 1] for i in range(0, len(ps), 2)]
            tot = ps[0] + acc0
            for p in perms:
                tot = tot + tot.at[p].get(mode="promise_in_bounds")
            pos = jnp.full((_L,), gout + e, jnp.int32)
            plsc.store_scatter(out_v, [pos], tot, mask=lane0)

    for slot in range(_NBUF):
        issue(slot, slot)

    def group_body(q, _):
        for slot in range(_NBUF):
            g = q * _NBUF + slot
            wait(slot)
            compute(g, slot)

            @pl.when(g + _NBUF < nchunk)
            def _():
                issue(g + _NBUF, slot)
            return_val = 0
        return return_val

    ngroups = nchunk // _NBUF
    lax.fori_loop(0, ngroups, group_body, 0)
    for g in range(ngroups * _NBUF, nchunk):
        slot = g % _NBUF
        wait(slot)
        compute(g, slot)
    pltpu.sync_copy(out_v, out_hbm.at[pl.ds(base, epw)])


def _edge_mlp(a, b, row, col, w2, b2):
    n, h = a.shape
    e = row.shape[0]
    info = plsc.get_sparse_core_info()
    nc, ns = info.num_cores, info.num_subcores
    nw = nc * ns
    epw = e // nw

    mesh = plsc.VectorSubcoreMesh(core_axis_name="c", subcore_axis_name="s")
    kern = pl.kernel(
        functools.partial(_edge_kernel, nc=nc, epw=epw, h=h),
        out_type=jax.ShapeDtypeStruct((e,), jnp.float32),
        mesh=mesh,
        compiler_params=pltpu.CompilerParams(needs_layout_passes=False),
        scratch_types=[
            pltpu.VMEM((epw,), jnp.int32),
            pltpu.VMEM((epw,), jnp.int32),
            pltpu.VMEM((_NBUF, _CHUNK, h), jnp.float32),
            pltpu.VMEM((_NBUF, _CHUNK, h), jnp.float32),
            pltpu.VMEM((h,), jnp.float32),
            pltpu.VMEM((_L,), jnp.float32),
            pltpu.VMEM((epw,), jnp.float32),
        ] + [pltpu.SemaphoreType.DMA] * (2 * _NBUF),
    )
    return kern(a, b, row, col, w2, b2)


def kernel(z, adj, W1, b1, W2, b2):
    a, b = _node_projections(z, W1, b1)
    b2v = jnp.broadcast_to(b2, (_L,))
    return _edge_mlp(a, b, adj[0], adj[1], W2.reshape(-1), b2v)


# speedup vs baseline: 1.0507x; 1.0507x over previous
import functools

import jax
import jax.numpy as jnp
from jax import lax
from jax.experimental import pallas as pl
from jax.experimental.pallas import tpu as pltpu
from jax.experimental.pallas import tpu_sc as plsc

_L = 16


def _proj_body(z_ref, w1a_ref, w1b_ref, b1_ref, a_ref, b_ref):
    z = z_ref[...]
    a_ref[...] = jnp.dot(z, w1a_ref[...], preferred_element_type=jnp.float32)
    b_ref[...] = (jnp.dot(z, w1b_ref[...], preferred_element_type=jnp.float32)
                  + b1_ref[...])


def _node_projections(z, W1, b1):
    n, h = z.shape
    blk = 1000
    grid = n // blk
    return pl.pallas_call(
        _proj_body,
        grid=(grid,),
        in_specs=[
            pl.BlockSpec((blk, h), lambda i: (i, 0)),
            pl.BlockSpec((h, h), lambda i: (0, 0)),
            pl.BlockSpec((h, h), lambda i: (0, 0)),
            pl.BlockSpec((1, h), lambda i: (0, 0)),
        ],
        out_specs=[
            pl.BlockSpec((blk, h), lambda i: (i, 0)),
            pl.BlockSpec((blk, h), lambda i: (i, 0)),
        ],
        out_shape=[
            jax.ShapeDtypeStruct((n, h), jnp.float32),
            jax.ShapeDtypeStruct((n, h), jnp.float32),
        ],
    )(z, W1[:h], W1[h:], b1.reshape(1, h))


_NBUF = 4
_CHUNK = 80


def _edge_kernel(a_hbm, b_hbm, row_hbm, col_hbm, w2_hbm, b2_hbm, out_hbm,
                 idx_row, idx_col, arows, brows, w2_v, b2_v, out_v,
                 *sems, nc, epw, h):
    cid = lax.axis_index("c")
    sid = lax.axis_index("s")
    wid = sid * nc + cid
    base = wid * epw
    nchunk = epw // _CHUNK
    sem_a = sems[:_NBUF]
    sem_b = sems[_NBUF:]

    pltpu.sync_copy(w2_hbm, w2_v)
    pltpu.sync_copy(b2_hbm, b2_v)
    pltpu.sync_copy(row_hbm.at[pl.ds(base, epw)], idx_row)
    pltpu.sync_copy(col_hbm.at[pl.ds(base, epw)], idx_col)


    lanes = lax.iota(jnp.int32, _L)
    lane0 = lanes == 0
    perms = [lanes ^ (1 << p) for p in range(4)]
    acc0 = jnp.where(lane0, b2_v[...], jnp.zeros((_L,), jnp.float32))
    w2s = [w2_v[pl.ds(s * _L, _L)] for s in range(h // _L)]

    def issue(g, slot):
        off = g * _CHUNK
        pltpu.async_copy(a_hbm.at[idx_row.at[pl.ds(off, _CHUNK)]],
                         arows.at[slot], sem_a[slot])
        pltpu.async_copy(b_hbm.at[idx_col.at[pl.ds(off, _CHUNK)]],
                         brows.at[slot], sem_b[slot])

    def wait(slot):
        pltpu.make_async_copy(a_hbm.at[pl.ds(0, _CHUNK)], arows.at[slot],
                              sem_a[slot]).wait()
        pltpu.make_async_copy(b_hbm.at[pl.ds(0, _CHUNK)], brows.at[slot],
                              sem_b[slot]).wait()

    def compute(g, slot):
        ar = arows.at[slot]
        br = brows.at[slot]
        gout = g * _CHUNK

        @plsc.parallel_loop(0, _CHUNK, unroll=4)
        def edge_body(e):
            ps = []
            for s in range(h // _L):
                av = ar[e, pl.ds(s * _L, _L)]
                bv = br[e, pl.ds(s * _L, _L)]
                m = jnp.maximum(av + bv, 0.0)
                ps.append(m * w2s[s])
            while len(ps) > 1:
                ps = [ps[i] + ps[i + 1] for i in range(0, len(ps), 2)]
            tot = ps[0] + acc0
            for p in perms:
                tot = tot + tot.at[p].get(mode="promise_in_bounds")
            pos = jnp.full((_L,), gout + e, jnp.int32)
            plsc.store_scatter(out_v, [pos], tot, mask=lane0)

    for slot in range(_NBUF):
        issue(slot, slot)

    def group_body(q, _):
        for slot in range(_NBUF):
            g = q * _NBUF + slot
            wait(slot)
            compute(g, slot)

            @pl.when(g + _NBUF < nchunk)
            def _():
                issue(g + _NBUF, slot)
            return_val = 0
        return return_val

    ngroups = nchunk // _NBUF
    lax.fori_loop(0, ngroups, group_body, 0)
    for g in range(ngroups * _NBUF, nchunk):
        slot = g % _NBUF
        wait(slot)
        compute(g, slot)
    pltpu.sync_copy(out_v, out_hbm.at[pl.ds(base, epw)])


def _edge_mlp(a, b, row, col, w2, b2):
    n, h = a.shape
    e = row.shape[0]
    info = plsc.get_sparse_core_info()
    nc, ns = info.num_cores, info.num_subcores
    nw = nc * ns
    epw = e // nw

    mesh = plsc.VectorSubcoreMesh(core_axis_name="c", subcore_axis_name="s")
    kern = pl.kernel(
        functools.partial(_edge_kernel, nc=nc, epw=epw, h=h),
        out_type=jax.ShapeDtypeStruct((e,), jnp.float32),
        mesh=mesh,
        compiler_params=pltpu.CompilerParams(needs_layout_passes=False),
        scratch_types=[
            pltpu.VMEM((epw,), jnp.int32),
            pltpu.VMEM((epw,), jnp.int32),
            pltpu.VMEM((_NBUF, _CHUNK, h), jnp.float32),
            pltpu.VMEM((_NBUF, _CHUNK, h), jnp.float32),
            pltpu.VMEM((h,), jnp.float32),
            pltpu.VMEM((_L,), jnp.float32),
            pltpu.VMEM((epw,), jnp.float32),
        ] + [pltpu.SemaphoreType.DMA] * (2 * _NBUF),
    )
    return kern(a, b, row, col, w2, b2)


def kernel(z, adj, W1, b1, W2, b2):
    a, b = _node_projections(z, W1, b1)
    b2v = jnp.broadcast_to(b2, (_L,))
    return _edge_mlp(a, b, adj[0], adj[1], W2.reshape(-1), b2v)
